# baseline (speedup 1.0000x reference)
import functools

import jax
import jax.numpy as jnp
from jax import lax
from jax.experimental import pallas as pl
from jax.experimental.pallas import tpu as pltpu
from jax.experimental.pallas import tpu_sc as plsc

N = 50000
E = 800000
HID = 64
EPS = 1e-5

NW = 32
R = 1568
NP = NW * R
CHUNK = 128
NCHUNKS = E // CHUNK
CPW = (NCHUNKS + NW - 1) // NW
CAP = CPW * CHUNK

BR = 1568
GRID = NP // BR

_MESH = plsc.VectorSubcoreMesh(core_axis_name="c", subcore_axis_name="s")
_SC_PARAMS = pltpu.CompilerParams(use_tc_tiling_on_sc=False,
                                  needs_layout_passes=False)


def _wid():
    return lax.axis_index("s") * 2 + lax.axis_index("c")


_LANE = lambda: lax.broadcasted_iota(jnp.int32, (16,), 0)


def _bucket_body(src_hbm, dst_hbm, bkt_hbm, cnt_hbm,
                 sbuf, dbuf, wbuf, pkbuf, posp, posx, cntbuf, sem):
    w = _wid()
    lane0 = _LANE() == 0
    for g in range(3):
        cntbuf[pl.ds(g * 16, 16)] = jnp.zeros((16,), jnp.int32)

    def chunk(j, _):
        c = w + NW * j

        @pl.when(c < NCHUNKS)
        def _():
            start = c * CHUNK
            pltpu.sync_copy(src_hbm.at[pl.ds(start, CHUNK)], sbuf)
            pltpu.sync_copy(dst_hbm.at[pl.ds(start, CHUNK)], dbuf)
            for g in range(CHUNK // 16):
                sl = pl.ds(g * 16, 16)
                s = sbuf[sl]
                d = dbuf[sl]
                sp = (s & 31) * R + (s >> 5)
                wbuf[sl] = d & 31
                pkbuf[sl] = sp | ((d >> 5) << 16)

            def edge(e, _):
                b = wbuf[pl.ds(e, 16)][0]
                cnt = cntbuf[pl.ds(b, 16)][0]
                pos = (w * NW + b) * CAP + cnt
                posp[pl.ds(e, 16)] = jnp.full((16,), pos, jnp.int32)
                cv = cntbuf[pl.ds(b, 16)]
                cntbuf[pl.ds(b, 16)] = jnp.where(lane0, cnt + 1, cv)
                return 0

            lax.fori_loop(0, CHUNK, edge, 0)
            for g in range(CHUNK // 16):
                sl = pl.ds(g * 16, 16)
                posx[sl] = posp[sl]
            pltpu.async_copy(pkbuf, bkt_hbm.at[posx], sem).wait()
        return 0

    lax.fori_loop(0, CPW, chunk, 0)
    pltpu.sync_copy(cntbuf.at[pl.ds(0, NW)], cnt_hbm.at[pl.ds(w * NW, NW)])


def _sc_bucket(src, dst):
    return pl.kernel(
        _bucket_body,
        out_type=[jax.ShapeDtypeStruct((NW * NW * CAP,), jnp.int32),
                  jax.ShapeDtypeStruct((NW * NW,), jnp.int32)],
        mesh=_MESH,
        compiler_params=_SC_PARAMS,
        scratch_types=[pltpu.VMEM((CHUNK,), jnp.int32),
                       pltpu.VMEM((CHUNK,), jnp.int32),
                       pltpu.VMEM((CHUNK + 16,), jnp.int32),
                       pltpu.VMEM((CHUNK,), jnp.int32),
                       pltpu.VMEM((CHUNK + 16,), jnp.int32),
                       pltpu.VMEM((CHUNK,), jnp.int32),
                       pltpu.VMEM((NW + 16,), jnp.int32),
                       pltpu.SemaphoreType.DMA],
    )(src, dst)


def _deg_body(bkt_hbm, cnt_hbm, deg_hbm, cntv, pkbuf, acc, tag, sem):
    w = _wid()
    lane0 = _LANE() == 0

    lane16 = _LANE()
    ones = jnp.ones((16,), jnp.float32)

    def zero(i, _):
        for u in range(8):
            acc[pl.ds(i * 128 + u * 16, 16)] = jnp.zeros((16,), jnp.float32)
        return 0
    lax.fori_loop(0, (R + 16) // 128, zero, 0)
    pltpu.sync_copy(cnt_hbm, cntv.at[pl.ds(0, NW * NW)])

    def writer(v, _):
        cnt = cntv[pl.ds(v * NW + w, 16)][0]

        def chunk(j, _):
            base = (v * NW + w) * CAP + j * CHUNK
            pltpu.sync_copy(bkt_hbm.at[pl.ds(base, CHUNK)],
                            pkbuf.at[pl.ds(0, CHUNK)])
            ne = jnp.minimum(cnt - j * CHUNK, CHUNK)

            def group(g, _):
                b16 = g * 16
                dlv = jnp.clip(pkbuf[pl.ds(b16, 16)] >> 16, 0, R - 1)
                emask = (lane16 + b16) < ne
                plsc.store_scatter(tag, [dlv], lane16, mask=emask)
                t = plsc.load_gather(tag, [dlv])
                ndup = plsc.all_reduce_population_count(
                    (t != lane16) & emask)[0]

                @pl.when(ndup == 0)
                def _():
                    plsc.addupdate_scatter(acc, [dlv], ones, mask=emask)

                @pl.when(ndup > 0)
                def _():
                    def edge(e, _):
                        dl = pkbuf[pl.ds(e, 16)][0] >> 16
                        a = acc[pl.ds(dl, 16)]
                        acc[pl.ds(dl, 16)] = jnp.where(lane0, a + 1.0, a)
                        return 0
                    lax.fori_loop(b16, jnp.minimum(b16 + 16, ne), edge, 0)
                return 0

            lax.fori_loop(0, (ne + 15) // 16, group, 0)
            return 0

        lax.fori_loop(0, (cnt + CHUNK - 1) // CHUNK, chunk, 0)
        return 0

    lax.fori_loop(0, NW, writer, 0)
    pltpu.sync_copy(acc.at[pl.ds(0, R)], deg_hbm.at[pl.ds(w * R, R)])


def _sc_deg(bkt, cnts):
    return pl.kernel(
        _deg_body,
        out_type=jax.ShapeDtypeStruct((NP,), jnp.float32),
        mesh=_MESH,
        compiler_params=_SC_PARAMS,
        scratch_types=[pltpu.VMEM((NW * NW + 16,), jnp.int32),
                       pltpu.VMEM((CHUNK + 16,), jnp.int32),
                       pltpu.VMEM((R + 16,), jnp.float32),
                       pltpu.VMEM((R,), jnp.int32),
                       pltpu.SemaphoreType.DMA],
    )(bkt, cnts)


def _prop_body(is_max, F, table_hbm, bkt_hbm, cnt_hbm, out_hbm,
               cntv, pkbuf, idxbuf, dlbuf, rows, acc, tag, sem):
    w = _wid()
    fill = jnp.full((16,), -3.4e38 if is_max else 0.0, jnp.float32)
    lane16 = _LANE()

    def zero(i, _):
        for u in range(8):
            acc[pl.ds(i * 128 + u * 16, 16)] = fill
        return 0
    lax.fori_loop(0, (R * F) // 128, zero, 0)
    pltpu.sync_copy(cnt_hbm, cntv.at[pl.ds(0, NW * NW)])

    def writer(v, _):
        cnt = cntv[pl.ds(v * NW + w, 16)][0]

        def chunk(j, _):
            base = (v * NW + w) * CAP + j * CHUNK
            pltpu.sync_copy(bkt_hbm.at[pl.ds(base, CHUNK)],
                            pkbuf.at[pl.ds(0, CHUNK)])
            for g in range(CHUNK // 16):
                sl = pl.ds(g * 16, 16)
                pk = pkbuf[sl]
                idxbuf[sl] = jnp.minimum(pk & 0xFFFF, NP - 1)
                dlbuf[sl] = jnp.clip(pk >> 16, 0, R - 1)
            pltpu.async_copy(table_hbm.at[idxbuf], rows, sem).wait()
            ne = jnp.minimum(cnt - j * CHUNK, CHUNK)

            def group(g, _):
                b16 = g * 16
                dlv = dlbuf[pl.ds(b16, 16)]
                ev = lane16 + b16
                emask = ev < ne
                aidx0 = dlv * F
                plsc.store_scatter(tag, [dlv], lane16, mask=emask)
                t = plsc.load_gather(tag, [dlv])
                ndup = plsc.all_reduce_population_count(
                    (t != lane16) & emask)[0]

                @pl.when(ndup == 0)
                def _():
                    for f in range(F):
                        fv = jnp.full((16,), f, jnp.int32)
                        val = plsc.load_gather(rows, [ev, fv])
                        if is_max:
                            a = plsc.load_gather(acc, [aidx0 + f])
                            plsc.store_scatter(acc, [aidx0 + f],
                                               jnp.maximum(a, val),
                                               mask=emask)
                        else:
                            plsc.addupdate_scatter(acc, [aidx0 + f], val,
                                                   mask=emask)

                @pl.when(ndup > 0)
                def _():
                    def edge(e, _):
                        dl = dlbuf[pl.ds(e, 16)][0]
                        for q in range(F // 16):
                            sl = pl.ds(q * 16, 16)
                            a = acc[pl.ds(dl * F + q * 16, 16)]
                            r = rows[e, sl]
                            acc[pl.ds(dl * F + q * 16, 16)] = (
                                jnp.maximum(a, r) if is_max else a + r)
                        return 0
                    lax.fori_loop(b16, jnp.minimum(b16 + 16, ne), edge, 0)
                return 0

            lax.fori_loop(0, (ne + 15) // 16, group, 0)
            return 0

        lax.fori_loop(0, (cnt + CHUNK - 1) // CHUNK, chunk, 0)
        return 0

    lax.fori_loop(0, NW, writer, 0)
    pltpu.sync_copy(acc, out_hbm.at[pl.ds(w * R * F, R * F)])


def _sc_prop(table, bkt, cnts, is_max):
    F = table.shape[1]
    out = pl.kernel(
        functools.partial(_prop_body, is_max, F),
        out_type=jax.ShapeDtypeStruct((NP * F,), jnp.float32),
        mesh=_MESH,
        compiler_params=_SC_PARAMS,
        scratch_types=[pltpu.VMEM((NW * NW + 16,), jnp.int32),
                       pltpu.VMEM((CHUNK,), jnp.int32),
                       pltpu.VMEM((CHUNK,), jnp.int32),
                       pltpu.VMEM((CHUNK + 16,), jnp.int32),
                       pltpu.VMEM((CHUNK, F), jnp.float32),
                       pltpu.VMEM((R * F,), jnp.float32),
                       pltpu.VMEM((R,), jnp.int32),
                       pltpu.SemaphoreType.DMA],
    )(table, bkt, cnts)
    return out.reshape(NP, F)


def _xperm_body(x_hbm, inv_hbm, out_hbm, idxbuf, rows, sem):
    w = _wid()
    nfull = R // CHUNK
    tail = R - nfull * CHUNK
    for j in range(nfull):
        pltpu.sync_copy(inv_hbm.at[pl.ds(w * R + j * CHUNK, CHUNK)], idxbuf)
        pltpu.async_copy(x_hbm.at[idxbuf], rows, sem).wait()
        pltpu.sync_copy(rows, out_hbm.at[pl.ds(w * R + j * CHUNK, CHUNK)])
    pltpu.sync_copy(inv_hbm.at[pl.ds(w * R + nfull * CHUNK, tail)],
                    idxbuf.at[pl.ds(0, tail)])
    pltpu.async_copy(x_hbm.at[idxbuf.at[pl.ds(0, tail)]],
                     rows.at[pl.ds(0, tail)], sem).wait()
    pltpu.sync_copy(rows.at[pl.ds(0, tail)],
                    out_hbm.at[pl.ds(w * R + nfull * CHUNK, tail)])


def _sc_xperm(x, invp):
    F = x.shape[1]
    return pl.kernel(
        _xperm_body,
        out_type=jax.ShapeDtypeStruct((NP, F), jnp.float32),
        mesh=_MESH,
        compiler_params=_SC_PARAMS,
        scratch_types=[pltpu.VMEM((CHUNK,), jnp.int32),
                       pltpu.VMEM((CHUNK, F), jnp.float32),
                       pltpu.SemaphoreType.DMA],
    )(x, invp)


def _nrm(deg):
    return lax.rsqrt(jnp.clip(deg, 1.0, None))


def _dot(a, b):
    return jax.lax.dot_general(a, b, (((1,), (0,)), ((), ())),
                               precision=jax.lax.Precision.HIGHEST)


def _row_spec(F):
    return pl.BlockSpec((BR, F), lambda i: (i, 0))


def _fix_spec(r, c):
    return pl.BlockSpec((r, c), lambda i: (0, 0))


def _scale_body(x_ref, deg_ref, y_ref):
    y_ref[...] = x_ref[...] * _nrm(deg_ref[...])


def _tc_scale(x, deg):
    F = x.shape[1]
    return pl.pallas_call(
        _scale_body, grid=(GRID,),
        in_specs=[_row_spec(F), _row_spec(1)],
        out_specs=_row_spec(F),
        out_shape=jax.ShapeDtypeStruct((NP, F), jnp.float32),
    )(x, deg)


def _mid_body(s_ref, deg_ref, x1_ref, y1_ref):
    nrm = _nrm(deg_ref[...])
    x1 = -(s_ref[...] * nrm)
    x1_ref[...] = x1
    y1_ref[...] = x1 * nrm


def _tc_mid(s0, deg):
    F = s0.shape[1]
    sh = jax.ShapeDtypeStruct((NP, F), jnp.float32)
    return pl.pallas_call(
        _mid_body, grid=(GRID,),
        in_specs=[_row_spec(F), _row_spec(1)],
        out_specs=[_row_spec(F), _row_spec(F)],
        out_shape=[sh, sh],
    )(s0, deg)


def _stats_tail(i, zm, zm_ref, ss_ref, sq_ref):
    zm_ref[...] = zm

    @pl.when(i == 0)
    def _():
        ss_ref[...] = jnp.zeros_like(ss_ref)
        sq_ref[...] = jnp.zeros_like(sq_ref)

    ss_ref[...] += jnp.sum(zm, axis=0, keepdims=True)
    sq_ref[...] += jnp.sum(zm * zm, axis=0, keepdims=True)


def _stats_cheb_body(h_ref, x1_ref, s1_ref, deg_ref, msk_ref, w_ref, b_ref,
                     zm_ref, ss_ref, sq_ref):
    F = h_ref.shape[1]
    nrm = _nrm(deg_ref[...])
    h = h_ref[...]
    x2 = -2.0 * (s1_ref[...] * nrm) - h
    W = w_ref[...]
    z = (_dot(h, W[:F]) + _dot(x1_ref[...], W[F:2 * F])
         + _dot(x2, W[2 * F:]) + b_ref[...])
    _stats_tail(pl.program_id(0), z * msk_ref[...], zm_ref, ss_ref, sq_ref)


def _tc_stats_cheb(h, x1, s1, deg, msk, W, b):
    F = h.shape[1]
    s64 = jax.ShapeDtypeStruct((1, HID), jnp.float32)
    return pl.pallas_call(
        _stats_cheb_body, grid=(GRID,),
        in_specs=[_row_spec(F), _row_spec(F), _row_spec(F), _row_spec(1),
                  _row_spec(1), _fix_spec(3 * F, HID), _fix_spec(1, HID)],
        out_specs=[_row_spec(HID), _fix_spec(1, HID), _fix_spec(1, HID)],
        out_shape=[jax.ShapeDtypeStruct((NP, HID), jnp.float32), s64, s64],
    )(h, x1, s1, deg, msk, W, b.reshape(1, HID))


def _stats_edge_body(m_ref, c_ref, deg_ref, msk_ref, zm_ref, ss_ref, sq_ref):
    z = jnp.where(deg_ref[...] > 0.0, m_ref[...] + c_ref[...], 0.0)
    _stats_tail(pl.program_id(0), z * msk_ref[...], zm_ref, ss_ref, sq_ref)


def _tc_stats_edge(m, c, deg, msk):
    s64 = jax.ShapeDtypeStruct((1, HID), jnp.float32)
    return pl.pallas_call(
        _stats_edge_body, grid=(GRID,),
        in_specs=[_row_spec(HID), _row_spec(HID), _row_spec(1), _row_spec(1)],
        out_specs=[_row_spec(HID), _fix_spec(1, HID), _fix_spec(1, HID)],
        out_shape=[jax.ShapeDtypeStruct((NP, HID), jnp.float32), s64, s64],
    )(m, c, deg, msk)


def _bn_h(zm_ref, ss_ref, sq_ref, g_ref, b_ref, msk_ref):
    m = ss_ref[...] * (1.0 / N)
    v = sq_ref[...] * (1.0 / N) - m * m
    h = (zm_ref[...] - m) * lax.rsqrt(v + EPS) * g_ref[...] + b_ref[...]
    return jnp.maximum(h, 0.0) * msk_ref[...]


def _apply_edge_body(zm_ref, ss_ref, sq_ref, g_ref, b_ref, msk_ref,
                     tw_ref, pw_ref, tb_ref, pb_ref, h_ref, a_ref, c_ref):
    h = _bn_h(zm_ref, ss_ref, sq_ref, g_ref, b_ref, msk_ref)
    a = _dot(h, tw_ref[...])
    h_ref[...] = h
    a_ref[...] = a
    c_ref[...] = _dot(h, pw_ref[...]) - a + tb_ref[...] + pb_ref[...]


def _tc_apply_edge(zm, ss, sq, g, b, msk, tw, pw, tb, pb):
    sh = jax.ShapeDtypeStruct((NP, HID), jnp.float32)
    return pl.pallas_call(
        _apply_edge_body, grid=(GRID,),
        in_specs=[_row_spec(HID), _fix_spec(1, HID), _fix_spec(1, HID),
                  _fix_spec(1, HID), _fix_spec(1, HID), _row_spec(1),
                  _fix_spec(HID, HID), _fix_spec(HID, HID),
                  _fix_spec(1, HID), _fix_spec(1, HID)],
        out_specs=[_row_spec(HID), _row_spec(HID), _row_spec(HID)],
        out_shape=[sh, sh, sh],
    )(zm, ss, sq, g.reshape(1, HID), b.reshape(1, HID), msk, tw, pw,
      tb.reshape(1, HID), pb.reshape(1, HID))


def _apply_cheb_body(zm_ref, ss_ref, sq_ref, g_ref, b_ref, msk_ref, deg_ref,
                     h_ref, y_ref):
    h = _bn_h(zm_ref, ss_ref, sq_ref, g_ref, b_ref, msk_ref)
    h_ref[...] = h
    y_ref[...] = h * _nrm(deg_ref[...])


def _tc_apply_cheb(zm, ss, sq, g, b, msk, deg):
    sh = jax.ShapeDtypeStruct((NP, HID), jnp.float32)
    return pl.pallas_call(
        _apply_cheb_body, grid=(GRID,),
        in_specs=[_row_spec(HID), _fix_spec(1, HID), _fix_spec(1, HID),
                  _fix_spec(1, HID), _fix_spec(1, HID), _row_spec(1),
                  _row_spec(1)],
        out_specs=[_row_spec(HID), _row_spec(HID)],
        out_shape=[sh, sh],
    )(zm, ss, sq, g.reshape(1, HID), b.reshape(1, HID), msk, deg)


def _apply_last_body(zm_ref, ss_ref, sq_ref, g_ref, b_ref, msk_ref, o_ref):
    h = _bn_h(zm_ref, ss_ref, sq_ref, g_ref, b_ref, msk_ref)
    i = pl.program_id(0)

    @pl.when(i == 0)
    def _():
        o_ref[...] = jnp.zeros_like(o_ref)

    o_ref[...] += jnp.sum(h, axis=0, keepdims=True) * (1.0 / N)


def _tc_apply_last(zm, ss, sq, g, b, msk):
    return pl.pallas_call(
        _apply_last_body, grid=(GRID,),
        in_specs=[_row_spec(HID), _fix_spec(1, HID), _fix_spec(1, HID),
                  _fix_spec(1, HID), _fix_spec(1, HID), _row_spec(1)],
        out_specs=_fix_spec(1, HID),
        out_shape=jax.ShapeDtypeStruct((1, HID), jnp.float32),
    )(zm, ss, sq, g.reshape(1, HID), b.reshape(1, HID), msk)


def _consts():
    p = jnp.arange(NP, dtype=jnp.int32)
    k = p % R
    w = p // R
    valid = k < (N - w + 31) // 32
    invp = jnp.where(valid, 32 * k + w, 0).astype(jnp.int32)
    vmask = valid.astype(jnp.float32).reshape(NP, 1)
    return invp, vmask


def kernel(x, edge_index, W1, b1, bn1_g, bn1_b, e1_tw, e1_tb, e1_pw, e1_pb,
           bne1_g, bne1_b, W2, b2, bn2_g, bn2_b, e2_tw, e2_tb, e2_pw, e2_pb,
           bne2_g, bne2_b, W3, b3, bn3_g, bn3_b):
    src = edge_index[0]
    dst = edge_index[1]
    invp, vmask = _consts()

    bkt, cnts = _sc_bucket(src, dst)
    deg = _sc_deg(bkt, cnts).reshape(NP, 1)
    xp = _sc_xperm(x, invp)

    def cheb_props(h_or_y0_pair, W, b, h_for_w0):
        y0 = h_or_y0_pair
        s0 = _sc_prop(y0, bkt, cnts, False)
        x1, y1 = _tc_mid(s0, deg)
        s1 = _sc_prop(y1, bkt, cnts, False)
        return _tc_stats_cheb(h_for_w0, x1, s1, deg, vmask, W, b)

    y0 = _tc_scale(xp, deg)
    zm, ss, sq = cheb_props(y0, W1, b1, xp)
    h, a, c = _tc_apply_edge(zm, ss, sq, bn1_g, bn1_b, vmask,
                             e1_tw, e1_pw, e1_tb, e1_pb)
    m = _sc_prop(a, bkt, cnts, True)
    zm, ss, sq = _tc_stats_edge(m, c, deg, vmask)
    h, y = _tc_apply_cheb(zm, ss, sq, bne1_g, bne1_b, vmask, deg)
    zm, ss, sq = cheb_props(y, W2, b2, h)
    h, a, c = _tc_apply_edge(zm, ss, sq, bn2_g, bn2_b, vmask,
                             e2_tw, e2_pw, e2_tb, e2_pb)
    m = _sc_prop(a, bkt, cnts, True)
    zm, ss, sq = _tc_stats_edge(m, c, deg, vmask)
    h, y = _tc_apply_cheb(zm, ss, sq, bne2_g, bne2_b, vmask, deg)
    zm, ss, sq = cheb_props(y, W3, b3, h)
    return _tc_apply_last(zm, ss, sq, bn3_g, bn3_b, vmask)


# speedup vs baseline: 1.4736x; 1.4736x over previous
import functools

import jax
import jax.numpy as jnp
from jax import lax
from jax.experimental import pallas as pl
from jax.experimental.pallas import tpu as pltpu
from jax.experimental.pallas import tpu_sc as plsc

N = 50000
E = 800000
HID = 64
EPS = 1e-5

NW = 32
R = 1568
NP = NW * R
CHUNK = 128
NCHUNKS = E // CHUNK
CPW = (NCHUNKS + NW - 1) // NW
CAP = CPW * CHUNK

BR = 1568
GRID = NP // BR

_MESH = plsc.VectorSubcoreMesh(core_axis_name="c", subcore_axis_name="s")
_SC_PARAMS = pltpu.CompilerParams(use_tc_tiling_on_sc=False,
                                  needs_layout_passes=False)


def _wid():
    return lax.axis_index("s") * 2 + lax.axis_index("c")


_LANE = lambda: lax.broadcasted_iota(jnp.int32, (16,), 0)


def _bucket_body(src_hbm, dst_hbm, bkt_hbm, cnt_hbm,
                 sbuf, dbuf, wbuf, pkbuf, posp, posx, cntbuf, sem):
    w = _wid()
    lane0 = _LANE() == 0
    for g in range(3):
        cntbuf[pl.ds(g * 16, 16)] = jnp.zeros((16,), jnp.int32)

    def chunk(j, _):
        c = w + NW * j

        @pl.when(c < NCHUNKS)
        def _():
            start = c * CHUNK
            pltpu.sync_copy(src_hbm.at[pl.ds(start, CHUNK)], sbuf)
            pltpu.sync_copy(dst_hbm.at[pl.ds(start, CHUNK)], dbuf)
            for g in range(CHUNK // 16):
                sl = pl.ds(g * 16, 16)
                s = sbuf[sl]
                d = dbuf[sl]
                sp = (s & 31) * R + (s >> 5)
                wbuf[sl] = d & 31
                pkbuf[sl] = sp | ((d >> 5) << 16)

            def edge(e, _):
                b = wbuf[pl.ds(e, 16)][0]
                cnt = cntbuf[pl.ds(b, 16)][0]
                pos = (w * NW + b) * CAP + cnt
                posp[pl.ds(e, 16)] = jnp.full((16,), pos, jnp.int32)
                cv = cntbuf[pl.ds(b, 16)]
                cntbuf[pl.ds(b, 16)] = jnp.where(lane0, cnt + 1, cv)
                return 0

            lax.fori_loop(0, CHUNK, edge, 0)
            for g in range(CHUNK // 16):
                sl = pl.ds(g * 16, 16)
                posx[sl] = posp[sl]
            pltpu.async_copy(pkbuf, bkt_hbm.at[posx], sem).wait()
        return 0

    lax.fori_loop(0, CPW, chunk, 0)
    pltpu.sync_copy(cntbuf.at[pl.ds(0, NW)], cnt_hbm.at[pl.ds(w * NW, NW)])


def _sc_bucket(src, dst):
    return pl.kernel(
        _bucket_body,
        out_type=[jax.ShapeDtypeStruct((NW * NW * CAP,), jnp.int32),
                  jax.ShapeDtypeStruct((NW * NW,), jnp.int32)],
        mesh=_MESH,
        compiler_params=_SC_PARAMS,
        scratch_types=[pltpu.VMEM((CHUNK,), jnp.int32),
                       pltpu.VMEM((CHUNK,), jnp.int32),
                       pltpu.VMEM((CHUNK + 16,), jnp.int32),
                       pltpu.VMEM((CHUNK,), jnp.int32),
                       pltpu.VMEM((CHUNK + 16,), jnp.int32),
                       pltpu.VMEM((CHUNK,), jnp.int32),
                       pltpu.VMEM((NW + 16,), jnp.int32),
                       pltpu.SemaphoreType.DMA],
    )(src, dst)


def _deg_body(bkt_hbm, cnt_hbm, deg_hbm, cntv, pkbuf, dlbuf, acc, sem):
    w = _wid()
    lane16 = _LANE()
    z16 = jnp.zeros((16,), jnp.float32)

    def zero(i, _):
        acc[pl.ds(i * 16, 16)] = z16
        return 0
    lax.fori_loop(0, (R + 16) // 16, zero, 0)
    e0 = jnp.where(lane16 == 0, 1.0, 0.0).astype(jnp.float32)
    pltpu.sync_copy(cnt_hbm, cntv.at[pl.ds(0, NW * NW)])

    def writer(v, _):
        cnt = cntv[pl.ds(v * NW + w, 16)][0]

        def chunk(j, _):
            base = (v * NW + w) * CAP + j * CHUNK
            pltpu.sync_copy(bkt_hbm.at[pl.ds(base, CHUNK)], pkbuf)
            ne = jnp.minimum(cnt - j * CHUNK, CHUNK)
            for g in range(CHUNK // 16):
                sl = pl.ds(g * 16, 16)
                dlbuf[sl] = jnp.clip(pkbuf[sl] >> 16, 0, R - 1)
            def edge(e, _):
                dl = dlbuf[pl.ds(e, 16)][0]
                plsc.addupdate(acc.at[pl.ds(dl, 16)], e0)
                return 0
            lax.fori_loop(0, ne, edge, 0)
            return 0

        lax.fori_loop(0, (cnt + CHUNK - 1) // CHUNK, chunk, 0)
        return 0

    lax.fori_loop(0, NW, writer, 0)
    pltpu.sync_copy(acc.at[pl.ds(0, R)], deg_hbm.at[pl.ds(w * R, R)])


def _sc_deg(bkt, cnts):
    return pl.kernel(
        _deg_body,
        out_type=jax.ShapeDtypeStruct((NP,), jnp.float32),
        mesh=_MESH,
        compiler_params=_SC_PARAMS,
        scratch_types=[pltpu.VMEM((NW * NW + 16,), jnp.int32),
                       pltpu.VMEM((CHUNK,), jnp.int32),
                       pltpu.VMEM((CHUNK + 16,), jnp.int32),
                       pltpu.VMEM((R + 16,), jnp.float32),
                       pltpu.SemaphoreType.DMA],
    )(bkt, cnts)


def _sum_body(F, table_hbm, bkt_hbm, cnt_hbm, out_hbm,
              cntv, pkbuf, idxbuf, dlbuf, rows, acc, sem):
    w = _wid()
    lane16 = _LANE()

    z16 = jnp.zeros((16,), jnp.float32)

    def zero(i, _):
        for u in range(8):
            acc[pl.ds(i * 128 + u * 16, 16)] = z16
        return 0
    lax.fori_loop(0, (R * F) // 128, zero, 0)
    pltpu.sync_copy(cnt_hbm, cntv.at[pl.ds(0, NW * NW)])

    def writer(v, _):
        cnt = cntv[pl.ds(v * NW + w, 16)][0]

        def chunk(j, _):
            base = (v * NW + w) * CAP + j * CHUNK
            pltpu.sync_copy(bkt_hbm.at[pl.ds(base, CHUNK)], pkbuf)
            ne = jnp.minimum(cnt - j * CHUNK, CHUNK)
            for g in range(CHUNK // 16):
                sl = pl.ds(g * 16, 16)
                pk = pkbuf[sl]
                idxbuf[sl] = jnp.minimum(pk & 0xFFFF, NP - 1)
                dlbuf[sl] = jnp.clip(pk >> 16, 0, R - 1)
            pltpu.async_copy(table_hbm.at[idxbuf], rows, sem).wait()

            def edge(e, _):
                b = dlbuf[pl.ds(e, 16)][0] * F
                for q in range(F // 16):
                    plsc.addupdate(acc.at[pl.ds(b + q * 16, 16)],
                                   rows[e, pl.ds(q * 16, 16)])
                return 0

            lax.fori_loop(0, ne, edge, 0)
            return 0

        lax.fori_loop(0, (cnt + CHUNK - 1) // CHUNK, chunk, 0)
        return 0

    lax.fori_loop(0, NW, writer, 0)
    pltpu.sync_copy(acc, out_hbm.at[pl.ds(w * R * F, R * F)])


def _max_body(F, table_hbm, bkt_hbm, cnt_hbm, out_hbm,
              cntv, pkbuf, idxbuf, dlbuf, rows, acc, sem):
    w = _wid()
    fill = jnp.full((16,), -3.4e38, jnp.float32)

    def zero(i, _):
        for u in range(8):
            acc[pl.ds(i * 128 + u * 16, 16)] = fill
        return 0
    lax.fori_loop(0, (R * F) // 128, zero, 0)
    pltpu.sync_copy(cnt_hbm, cntv.at[pl.ds(0, NW * NW)])

    def writer(v, _):
        cnt = cntv[pl.ds(v * NW + w, 16)][0]

        def chunk(j, _):
            base = (v * NW + w) * CAP + j * CHUNK
            pltpu.sync_copy(bkt_hbm.at[pl.ds(base, CHUNK)], pkbuf)
            for g in range(CHUNK // 16):
                sl = pl.ds(g * 16, 16)
                pk = pkbuf[sl]
                idxbuf[sl] = jnp.minimum(pk & 0xFFFF, NP - 1)
                dlbuf[sl] = jnp.clip(pk >> 16, 0, R - 1)
            pltpu.async_copy(table_hbm.at[idxbuf], rows, sem).wait()
            ne = jnp.minimum(cnt - j * CHUNK, CHUNK)

            def edge(e, _):
                dl = dlbuf[pl.ds(e, 16)][0]
                for q in range(F // 16):
                    sl = pl.ds(q * 16, 16)
                    a = acc[pl.ds(dl * F + q * 16, 16)]
                    acc[pl.ds(dl * F + q * 16, 16)] = (
                        jnp.maximum(a, rows[e, sl]))
                return 0

            lax.fori_loop(0, ne, edge, 0)
            return 0

        lax.fori_loop(0, (cnt + CHUNK - 1) // CHUNK, chunk, 0)
        return 0

    lax.fori_loop(0, NW, writer, 0)
    pltpu.sync_copy(acc, out_hbm.at[pl.ds(w * R * F, R * F)])


def _sc_prop(table, bkt, cnts, is_max):
    F = table.shape[1]
    common = [pltpu.VMEM((NW * NW + 16,), jnp.int32),
              pltpu.VMEM((CHUNK,), jnp.int32),
              pltpu.VMEM((CHUNK,), jnp.int32),
              pltpu.VMEM((CHUNK + 16,), jnp.int32),
              pltpu.VMEM((CHUNK, F), jnp.float32)]
    if is_max:
        out = pl.kernel(
            functools.partial(_max_body, F),
            out_type=jax.ShapeDtypeStruct((NP * F,), jnp.float32),
            mesh=_MESH,
            compiler_params=_SC_PARAMS,
            scratch_types=common + [pltpu.VMEM((R * F,), jnp.float32),
                                    pltpu.SemaphoreType.DMA],
        )(table, bkt, cnts)
        return out.reshape(NP, F)
    out = pl.kernel(
        functools.partial(_sum_body, F),
        out_type=jax.ShapeDtypeStruct((NP * F,), jnp.float32),
        mesh=_MESH,
        compiler_params=_SC_PARAMS,
        scratch_types=common + [pltpu.VMEM((R * F,), jnp.float32),
                                pltpu.SemaphoreType.DMA],
    )(table, bkt, cnts)
    return out.reshape(NP, F)


def _xperm_body(x_hbm, inv_hbm, out_hbm, idxbuf, rows, sem):
    w = _wid()
    nfull = R // CHUNK
    tail = R - nfull * CHUNK
    for j in range(nfull):
        pltpu.sync_copy(inv_hbm.at[pl.ds(w * R + j * CHUNK, CHUNK)], idxbuf)
        pltpu.async_copy(x_hbm.at[idxbuf], rows, sem).wait()
        pltpu.sync_copy(rows, out_hbm.at[pl.ds(w * R + j * CHUNK, CHUNK)])
    pltpu.sync_copy(inv_hbm.at[pl.ds(w * R + nfull * CHUNK, tail)],
                    idxbuf.at[pl.ds(0, tail)])
    pltpu.async_copy(x_hbm.at[idxbuf.at[pl.ds(0, tail)]],
                     rows.at[pl.ds(0, tail)], sem).wait()
    pltpu.sync_copy(rows.at[pl.ds(0, tail)],
                    out_hbm.at[pl.ds(w * R + nfull * CHUNK, tail)])


def _sc_xperm(x, invp):
    F = x.shape[1]
    return pl.kernel(
        _xperm_body,
        out_type=jax.ShapeDtypeStruct((NP, F), jnp.float32),
        mesh=_MESH,
        compiler_params=_SC_PARAMS,
        scratch_types=[pltpu.VMEM((CHUNK,), jnp.int32),
                       pltpu.VMEM((CHUNK, F), jnp.float32),
                       pltpu.SemaphoreType.DMA],
    )(x, invp)


def _nrm(deg):
    return lax.rsqrt(jnp.clip(deg, 1.0, None))


def _dot(a, b):
    return jax.lax.dot_general(a, b, (((1,), (0,)), ((), ())),
                               precision=jax.lax.Precision.HIGHEST)


def _row_spec(F):
    return pl.BlockSpec((BR, F), lambda i: (i, 0))


def _fix_spec(r, c):
    return pl.BlockSpec((r, c), lambda i: (0, 0))


def _scale_body(x_ref, deg_ref, y_ref):
    y_ref[...] = x_ref[...] * _nrm(deg_ref[...])


def _tc_scale(x, deg):
    F = x.shape[1]
    return pl.pallas_call(
        _scale_body, grid=(GRID,),
        in_specs=[_row_spec(F), _row_spec(1)],
        out_specs=_row_spec(F),
        out_shape=jax.ShapeDtypeStruct((NP, F), jnp.float32),
    )(x, deg)


def _mid_body(s_ref, deg_ref, x1_ref, y1_ref):
    nrm = _nrm(deg_ref[...])
    x1 = -(s_ref[...] * nrm)
    x1_ref[...] = x1
    y1_ref[...] = x1 * nrm


def _tc_mid(s0, deg):
    F = s0.shape[1]
    sh = jax.ShapeDtypeStruct((NP, F), jnp.float32)
    return pl.pallas_call(
        _mid_body, grid=(GRID,),
        in_specs=[_row_spec(F), _row_spec(1)],
        out_specs=[_row_spec(F), _row_spec(F)],
        out_shape=[sh, sh],
    )(s0, deg)


def _stats_tail(i, zm, zm_ref, ss_ref, sq_ref):
    zm_ref[...] = zm

    @pl.when(i == 0)
    def _():
        ss_ref[...] = jnp.zeros_like(ss_ref)
        sq_ref[...] = jnp.zeros_like(sq_ref)

    ss_ref[...] += jnp.sum(zm, axis=0, keepdims=True)
    sq_ref[...] += jnp.sum(zm * zm, axis=0, keepdims=True)


def _stats_cheb_body(h_ref, x1_ref, s1_ref, deg_ref, msk_ref, w_ref, b_ref,
                     zm_ref, ss_ref, sq_ref):
    F = h_ref.shape[1]
    nrm = _nrm(deg_ref[...])
    h = h_ref[...]
    x2 = -2.0 * (s1_ref[...] * nrm) - h
    W = w_ref[...]
    z = (_dot(h, W[:F]) + _dot(x1_ref[...], W[F:2 * F])
         + _dot(x2, W[2 * F:]) + b_ref[...])
    _stats_tail(pl.program_id(0), z * msk_ref[...], zm_ref, ss_ref, sq_ref)


def _tc_stats_cheb(h, x1, s1, deg, msk, W, b):
    F = h.shape[1]
    s64 = jax.ShapeDtypeStruct((1, HID), jnp.float32)
    return pl.pallas_call(
        _stats_cheb_body, grid=(GRID,),
        in_specs=[_row_spec(F), _row_spec(F), _row_spec(F), _row_spec(1),
                  _row_spec(1), _fix_spec(3 * F, HID), _fix_spec(1, HID)],
        out_specs=[_row_spec(HID), _fix_spec(1, HID), _fix_spec(1, HID)],
        out_shape=[jax.ShapeDtypeStruct((NP, HID), jnp.float32), s64, s64],
    )(h, x1, s1, deg, msk, W, b.reshape(1, HID))


def _stats_edge_body(m_ref, c_ref, deg_ref, msk_ref, zm_ref, ss_ref, sq_ref):
    z = jnp.where(deg_ref[...] > 0.0, m_ref[...] + c_ref[...], 0.0)
    _stats_tail(pl.program_id(0), z * msk_ref[...], zm_ref, ss_ref, sq_ref)


def _tc_stats_edge(m, c, deg, msk):
    s64 = jax.ShapeDtypeStruct((1, HID), jnp.float32)
    return pl.pallas_call(
        _stats_edge_body, grid=(GRID,),
        in_specs=[_row_spec(HID), _row_spec(HID), _row_spec(1), _row_spec(1)],
        out_specs=[_row_spec(HID), _fix_spec(1, HID), _fix_spec(1, HID)],
        out_shape=[jax.ShapeDtypeStruct((NP, HID), jnp.float32), s64, s64],
    )(m, c, deg, msk)


def _bn_h(zm_ref, ss_ref, sq_ref, g_ref, b_ref, msk_ref):
    m = ss_ref[...] * (1.0 / N)
    v = sq_ref[...] * (1.0 / N) - m * m
    h = (zm_ref[...] - m) * lax.rsqrt(v + EPS) * g_ref[...] + b_ref[...]
    return jnp.maximum(h, 0.0) * msk_ref[...]


def _apply_edge_body(zm_ref, ss_ref, sq_ref, g_ref, b_ref, msk_ref,
                     tw_ref, pw_ref, tb_ref, pb_ref, h_ref, a_ref, c_ref):
    h = _bn_h(zm_ref, ss_ref, sq_ref, g_ref, b_ref, msk_ref)
    a = _dot(h, tw_ref[...])
    h_ref[...] = h
    a_ref[...] = a
    c_ref[...] = _dot(h, pw_ref[...]) - a + tb_ref[...] + pb_ref[...]


def _tc_apply_edge(zm, ss, sq, g, b, msk, tw, pw, tb, pb):
    sh = jax.ShapeDtypeStruct((NP, HID), jnp.float32)
    return pl.pallas_call(
        _apply_edge_body, grid=(GRID,),
        in_specs=[_row_spec(HID), _fix_spec(1, HID), _fix_spec(1, HID),
                  _fix_spec(1, HID), _fix_spec(1, HID), _row_spec(1),
                  _fix_spec(HID, HID), _fix_spec(HID, HID),
                  _fix_spec(1, HID), _fix_spec(1, HID)],
        out_specs=[_row_spec(HID), _row_spec(HID), _row_spec(HID)],
        out_shape=[sh, sh, sh],
    )(zm, ss, sq, g.reshape(1, HID), b.reshape(1, HID), msk, tw, pw,
      tb.reshape(1, HID), pb.reshape(1, HID))


def _apply_cheb_body(zm_ref, ss_ref, sq_ref, g_ref, b_ref, msk_ref, deg_ref,
                     h_ref, y_ref):
    h = _bn_h(zm_ref, ss_ref, sq_ref, g_ref, b_ref, msk_ref)
    h_ref[...] = h
    y_ref[...] = h * _nrm(deg_ref[...])


def _tc_apply_cheb(zm, ss, sq, g, b, msk, deg):
    sh = jax.ShapeDtypeStruct((NP, HID), jnp.float32)
    return pl.pallas_call(
        _apply_cheb_body, grid=(GRID,),
        in_specs=[_row_spec(HID), _fix_spec(1, HID), _fix_spec(1, HID),
                  _fix_spec(1, HID), _fix_spec(1, HID), _row_spec(1),
                  _row_spec(1)],
        out_specs=[_row_spec(HID), _row_spec(HID)],
        out_shape=[sh, sh],
    )(zm, ss, sq, g.reshape(1, HID), b.reshape(1, HID), msk, deg)


def _apply_last_body(zm_ref, ss_ref, sq_ref, g_ref, b_ref, msk_ref, o_ref):
    h = _bn_h(zm_ref, ss_ref, sq_ref, g_ref, b_ref, msk_ref)
    i = pl.program_id(0)

    @pl.when(i == 0)
    def _():
        o_ref[...] = jnp.zeros_like(o_ref)

    o_ref[...] += jnp.sum(h, axis=0, keepdims=True) * (1.0 / N)


def _tc_apply_last(zm, ss, sq, g, b, msk):
    return pl.pallas_call(
        _apply_last_body, grid=(GRID,),
        in_specs=[_row_spec(HID), _fix_spec(1, HID), _fix_spec(1, HID),
                  _fix_spec(1, HID), _fix_spec(1, HID), _row_spec(1)],
        out_specs=_fix_spec(1, HID),
        out_shape=jax.ShapeDtypeStruct((1, HID), jnp.float32),
    )(zm, ss, sq, g.reshape(1, HID), b.reshape(1, HID), msk)


def _consts():
    p = jnp.arange(NP, dtype=jnp.int32)
    k = p % R
    w = p // R
    valid = k < (N - w + 31) // 32
    invp = jnp.where(valid, 32 * k + w, 0).astype(jnp.int32)
    vmask = valid.astype(jnp.float32).reshape(NP, 1)
    return invp, vmask


def kernel(x, edge_index, W1, b1, bn1_g, bn1_b, e1_tw, e1_tb, e1_pw, e1_pb,
           bne1_g, bne1_b, W2, b2, bn2_g, bn2_b, e2_tw, e2_tb, e2_pw, e2_pb,
           bne2_g, bne2_b, W3, b3, bn3_g, bn3_b):
    src = edge_index[0]
    dst = edge_index[1]
    invp, vmask = _consts()

    bkt, cnts = _sc_bucket(src, dst)
    deg = _sc_deg(bkt, cnts).reshape(NP, 1)
    xp = _sc_xperm(x, invp)

    def cheb_props(h_or_y0_pair, W, b, h_for_w0):
        y0 = h_or_y0_pair
        s0 = _sc_prop(y0, bkt, cnts, False)
        x1, y1 = _tc_mid(s0, deg)
        s1 = _sc_prop(y1, bkt, cnts, False)
        return _tc_stats_cheb(h_for_w0, x1, s1, deg, vmask, W, b)

    y0 = _tc_scale(xp, deg)
    zm, ss, sq = cheb_props(y0, W1, b1, xp)
    h, a, c = _tc_apply_edge(zm, ss, sq, bn1_g, bn1_b, vmask,
                             e1_tw, e1_pw, e1_tb, e1_pb)
    m = _sc_prop(a, bkt, cnts, True)
    zm, ss, sq = _tc_stats_edge(m, c, deg, vmask)
    h, y = _tc_apply_cheb(zm, ss, sq, bne1_g, bne1_b, vmask, deg)
    zm, ss, sq = cheb_props(y, W2, b2, h)
    h, a, c = _tc_apply_edge(zm, ss, sq, bn2_g, bn2_b, vmask,
                             e2_tw, e2_pw, e2_tb, e2_pb)
    m = _sc_prop(a, bkt, cnts, True)
    zm, ss, sq = _tc_stats_edge(m, c, deg, vmask)
    h, y = _tc_apply_cheb(zm, ss, sq, bne2_g, bne2_b, vmask, deg)
    zm, ss, sq = cheb_props(y, W3, b3, h)
    return _tc_apply_last(zm, ss, sq, bn3_g, bn3_b, vmask)


# speedup vs baseline: 2.2942x; 1.5569x over previous
import functools

import jax
import jax.numpy as jnp
from jax import lax
from jax.experimental import pallas as pl
from jax.experimental.pallas import tpu as pltpu
from jax.experimental.pallas import tpu_sc as plsc

N = 50000
E = 800000
HID = 64
EPS = 1e-5

NW = 32
R = 1568
NP = NW * R
CHUNK = 128
NCHUNKS = E // CHUNK
CPW = (NCHUNKS + NW - 1) // NW
CAP = CPW * CHUNK

BR = 1568
GRID = NP // BR

_MESH = plsc.VectorSubcoreMesh(core_axis_name="c", subcore_axis_name="s")
_SC_PARAMS = pltpu.CompilerParams(use_tc_tiling_on_sc=False,
                                  needs_layout_passes=False)


def _wid():
    return lax.axis_index("s") * 2 + lax.axis_index("c")


_LANE = lambda: lax.broadcasted_iota(jnp.int32, (16,), 0)


def _bucket_body(src_hbm, dst_hbm, bkt_hbm, cnt_hbm,
                 sbuf, dbuf, wbuf, pkbuf, posp, posx, cntbuf, sem):
    w = _wid()
    lane0 = _LANE() == 0
    for g in range(3):
        cntbuf[pl.ds(g * 16, 16)] = jnp.zeros((16,), jnp.int32)

    def chunk(j, _):
        c = w + NW * j

        @pl.when(c < NCHUNKS)
        def _():
            start = c * CHUNK
            pltpu.sync_copy(src_hbm.at[pl.ds(start, CHUNK)], sbuf)
            pltpu.sync_copy(dst_hbm.at[pl.ds(start, CHUNK)], dbuf)
            for g in range(CHUNK // 16):
                sl = pl.ds(g * 16, 16)
                s = sbuf[sl]
                d = dbuf[sl]
                sp = (s & 31) * R + (s >> 5)
                wbuf[sl] = d & 31
                pkbuf[sl] = sp | ((d >> 5) << 16)

            def edge(e, _):
                b = wbuf[pl.ds(e, 16)][0]
                cnt = cntbuf[pl.ds(b, 16)][0]
                pos = (w * NW + b) * CAP + cnt
                posp[pl.ds(e, 16)] = jnp.full((16,), pos, jnp.int32)
                cv = cntbuf[pl.ds(b, 16)]
                cntbuf[pl.ds(b, 16)] = jnp.where(lane0, cnt + 1, cv)
                return 0

            lax.fori_loop(0, CHUNK, edge, 0)
            for g in range(CHUNK // 16):
                sl = pl.ds(g * 16, 16)
                posx[sl] = posp[sl]
            pltpu.async_copy(pkbuf, bkt_hbm.at[posx], sem).wait()
        return 0

    lax.fori_loop(0, CPW, chunk, 0)
    pltpu.sync_copy(cntbuf.at[pl.ds(0, NW)], cnt_hbm.at[pl.ds(w * NW, NW)])


def _sc_bucket(src, dst):
    return pl.kernel(
        _bucket_body,
        out_type=[jax.ShapeDtypeStruct((NW * NW * CAP,), jnp.int32),
                  jax.ShapeDtypeStruct((NW * NW,), jnp.int32)],
        mesh=_MESH,
        compiler_params=_SC_PARAMS,
        scratch_types=[pltpu.VMEM((CHUNK,), jnp.int32),
                       pltpu.VMEM((CHUNK,), jnp.int32),
                       pltpu.VMEM((CHUNK + 16,), jnp.int32),
                       pltpu.VMEM((CHUNK,), jnp.int32),
                       pltpu.VMEM((CHUNK + 16,), jnp.int32),
                       pltpu.VMEM((CHUNK,), jnp.int32),
                       pltpu.VMEM((NW + 16,), jnp.int32),
                       pltpu.SemaphoreType.DMA],
    )(src, dst)


def _deg_body(bkt_hbm, cnt_hbm, deg_hbm, cntv, pkbuf, dlbuf, acc, sem):
    w = _wid()
    lane16 = _LANE()
    z16 = jnp.zeros((16,), jnp.float32)

    def zero(i, _):
        for u in range(8):
            acc[pl.ds(i * 128 + u * 16, 16)] = z16
        return 0
    lax.fori_loop(0, R // 8, zero, 0)
    e0 = jnp.where(lane16 == 0, 1.0, 0.0).astype(jnp.float32)
    pltpu.sync_copy(cnt_hbm, cntv.at[pl.ds(0, NW * NW)])

    def writer(v, _):
        cnt = cntv[pl.ds(v * NW + w, 16)][0]

        def chunk(j, _):
            base = (v * NW + w) * CAP + j * CHUNK
            pltpu.sync_copy(bkt_hbm.at[pl.ds(base, CHUNK)], pkbuf)
            ne = jnp.minimum(cnt - j * CHUNK, CHUNK)
            for g in range(CHUNK // 16):
                sl = pl.ds(g * 16, 16)
                ok = (lane16 + g * 16) < ne
                dlbuf[sl] = jnp.where(
                    ok, jnp.clip(pkbuf[sl] >> 16, 0, R - 1), R - 1)

            def grp(g, _):
                dlv = dlbuf[pl.ds(g * 16, 16)]
                for l in range(16):
                    plsc.addupdate(acc.at[pl.ds(dlv[l] * 16, 16)], e0)
                return 0

            lax.fori_loop(0, CHUNK // 16, grp, 0)
            return 0

        lax.fori_loop(0, (cnt + CHUNK - 1) // CHUNK, chunk, 0)
        return 0

    lax.fori_loop(0, NW, writer, 0)
    pltpu.sync_copy(acc, deg_hbm.at[pl.ds(w * R * 16, R * 16)])


def _sc_deg(bkt, cnts):
    out = pl.kernel(
        _deg_body,
        out_type=jax.ShapeDtypeStruct((NP * 16,), jnp.float32),
        mesh=_MESH,
        compiler_params=_SC_PARAMS,
        scratch_types=[pltpu.VMEM((NW * NW + 16,), jnp.int32),
                       pltpu.VMEM((CHUNK,), jnp.int32),
                       pltpu.VMEM((CHUNK,), jnp.int32),
                       pltpu.VMEM((R * 16,), jnp.float32),
                       pltpu.SemaphoreType.DMA],
    )(bkt, cnts)
    return out.reshape(NP, 16)[:, 0]


def _sum_body(F, table_hbm, bkt_hbm, cnt_hbm, out_hbm,
              cntv, pkbuf, idxbuf, dlbuf, rows, acc, sem):
    w = _wid()
    lane16 = _LANE()

    z16 = jnp.zeros((16,), jnp.float32)

    def zero(i, _):
        for u in range(8):
            acc[pl.ds(i * 128 + u * 16, 16)] = z16
        return 0
    lax.fori_loop(0, (R * F) // 128, zero, 0)
    pltpu.sync_copy(cnt_hbm, cntv.at[pl.ds(0, NW * NW)])

    def writer(v, _):
        cnt = cntv[pl.ds(v * NW + w, 16)][0]

        def chunk(j, _):
            base = (v * NW + w) * CAP + j * CHUNK
            pltpu.sync_copy(bkt_hbm.at[pl.ds(base, CHUNK)], pkbuf)
            ne = jnp.minimum(cnt - j * CHUNK, CHUNK)
            for g in range(CHUNK // 16):
                sl = pl.ds(g * 16, 16)
                pk = pkbuf[sl]
                ok = (lane16 + g * 16) < ne
                idxbuf[sl] = jnp.minimum(pk & 0xFFFF, NP - 1)
                dlbuf[sl] = jnp.where(ok, jnp.clip(pk >> 16, 0, R - 1),
                                      R - 1)
            pltpu.async_copy(table_hbm.at[idxbuf], rows, sem).wait()

            def grp(g, _):
                dlv = dlbuf[pl.ds(g * 16, 16)]
                for l in range(16):
                    b = dlv[l] * F
                    for q in range(F // 16):
                        plsc.addupdate(acc.at[pl.ds(b + q * 16, 16)],
                                       rows[g * 16 + l, pl.ds(q * 16, 16)])
                return 0

            lax.fori_loop(0, CHUNK // 16, grp, 0)
            return 0

        lax.fori_loop(0, (cnt + CHUNK - 1) // CHUNK, chunk, 0)
        return 0

    lax.fori_loop(0, NW, writer, 0)
    pltpu.sync_copy(acc, out_hbm.at[pl.ds(w * R * F, R * F)])


def _max_body(F, table_hbm, bkt_hbm, cnt_hbm, out_hbm,
              cntv, pkbuf, idxbuf, dlbuf, rows, acc, sem):
    w = _wid()
    lane16 = _LANE()
    fill = jnp.full((16,), -3.4e38, jnp.float32)

    def zero(i, _):
        for u in range(8):
            acc[pl.ds(i * 128 + u * 16, 16)] = fill
        return 0
    lax.fori_loop(0, (R * F) // 128, zero, 0)
    pltpu.sync_copy(cnt_hbm, cntv.at[pl.ds(0, NW * NW)])

    def writer(v, _):
        cnt = cntv[pl.ds(v * NW + w, 16)][0]

        def chunk(j, _):
            base = (v * NW + w) * CAP + j * CHUNK
            pltpu.sync_copy(bkt_hbm.at[pl.ds(base, CHUNK)], pkbuf)
            ne = jnp.minimum(cnt - j * CHUNK, CHUNK)
            for g in range(CHUNK // 16):
                sl = pl.ds(g * 16, 16)
                pk = pkbuf[sl]
                ok = (lane16 + g * 16) < ne
                idxbuf[sl] = jnp.minimum(pk & 0xFFFF, NP - 1)
                dlbuf[sl] = jnp.where(ok, jnp.clip(pk >> 16, 0, R - 1),
                                      R - 1)
            pltpu.async_copy(table_hbm.at[idxbuf], rows, sem).wait()

            def grp(g, _):
                dlv = dlbuf[pl.ds(g * 16, 16)]
                for l in range(16):
                    b = dlv[l] * F
                    for q in range(F // 16):
                        a = acc[pl.ds(b + q * 16, 16)]
                        acc[pl.ds(b + q * 16, 16)] = jnp.maximum(
                            a, rows[g * 16 + l, pl.ds(q * 16, 16)])
                return 0

            lax.fori_loop(0, CHUNK // 16, grp, 0)
            return 0

        lax.fori_loop(0, (cnt + CHUNK - 1) // CHUNK, chunk, 0)
        return 0

    lax.fori_loop(0, NW, writer, 0)
    pltpu.sync_copy(acc, out_hbm.at[pl.ds(w * R * F, R * F)])


def _sc_prop(table, bkt, cnts, is_max):
    F = table.shape[1]
    common = [pltpu.VMEM((NW * NW + 16,), jnp.int32),
              pltpu.VMEM((CHUNK,), jnp.int32),
              pltpu.VMEM((CHUNK,), jnp.int32),
              pltpu.VMEM((CHUNK + 16,), jnp.int32),
              pltpu.VMEM((CHUNK, F), jnp.float32)]
    if is_max:
        out = pl.kernel(
            functools.partial(_max_body, F),
            out_type=jax.ShapeDtypeStruct((NP * F,), jnp.float32),
            mesh=_MESH,
            compiler_params=_SC_PARAMS,
            scratch_types=common + [pltpu.VMEM((R * F,), jnp.float32),
                                    pltpu.SemaphoreType.DMA],
        )(table, bkt, cnts)
        return out.reshape(NP, F)
    out = pl.kernel(
        functools.partial(_sum_body, F),
        out_type=jax.ShapeDtypeStruct((NP * F,), jnp.float32),
        mesh=_MESH,
        compiler_params=_SC_PARAMS,
        scratch_types=common + [pltpu.VMEM((R * F,), jnp.float32),
                                pltpu.SemaphoreType.DMA],
    )(table, bkt, cnts)
    return out.reshape(NP, F)


def _xperm_body(x_hbm, inv_hbm, out_hbm, idxbuf, rows, sem):
    w = _wid()
    nfull = R // CHUNK
    tail = R - nfull * CHUNK
    for j in range(nfull):
        pltpu.sync_copy(inv_hbm.at[pl.ds(w * R + j * CHUNK, CHUNK)], idxbuf)
        pltpu.async_copy(x_hbm.at[idxbuf], rows, sem).wait()
        pltpu.sync_copy(rows, out_hbm.at[pl.ds(w * R + j * CHUNK, CHUNK)])
    pltpu.sync_copy(inv_hbm.at[pl.ds(w * R + nfull * CHUNK, tail)],
                    idxbuf.at[pl.ds(0, tail)])
    pltpu.async_copy(x_hbm.at[idxbuf.at[pl.ds(0, tail)]],
                     rows.at[pl.ds(0, tail)], sem).wait()
    pltpu.sync_copy(rows.at[pl.ds(0, tail)],
                    out_hbm.at[pl.ds(w * R + nfull * CHUNK, tail)])


def _sc_xperm(x, invp):
    F = x.shape[1]
    return pl.kernel(
        _xperm_body,
        out_type=jax.ShapeDtypeStruct((NP, F), jnp.float32),
        mesh=_MESH,
        compiler_params=_SC_PARAMS,
        scratch_types=[pltpu.VMEM((CHUNK,), jnp.int32),
                       pltpu.VMEM((CHUNK, F), jnp.float32),
                       pltpu.SemaphoreType.DMA],
    )(x, invp)


def _nrm(deg):
    return lax.rsqrt(jnp.clip(deg, 1.0, None))


def _dot(a, b):
    return jax.lax.dot_general(a, b, (((1,), (0,)), ((), ())),
                               precision=jax.lax.Precision.HIGHEST)


def _row_spec(F):
    return pl.BlockSpec((BR, F), lambda i: (i, 0))


def _fix_spec(r, c):
    return pl.BlockSpec((r, c), lambda i: (0, 0))


def _scale_body(x_ref, deg_ref, y_ref):
    y_ref[...] = x_ref[...] * _nrm(deg_ref[...])


def _tc_scale(x, deg):
    F = x.shape[1]
    return pl.pallas_call(
        _scale_body, grid=(GRID,),
        in_specs=[_row_spec(F), _row_spec(1)],
        out_specs=_row_spec(F),
        out_shape=jax.ShapeDtypeStruct((NP, F), jnp.float32),
    )(x, deg)


def _mid_body(s_ref, deg_ref, x1_ref, y1_ref):
    nrm = _nrm(deg_ref[...])
    x1 = -(s_ref[...] * nrm)
    x1_ref[...] = x1
    y1_ref[...] = x1 * nrm


def _tc_mid(s0, deg):
    F = s0.shape[1]
    sh = jax.ShapeDtypeStruct((NP, F), jnp.float32)
    return pl.pallas_call(
        _mid_body, grid=(GRID,),
        in_specs=[_row_spec(F), _row_spec(1)],
        out_specs=[_row_spec(F), _row_spec(F)],
        out_shape=[sh, sh],
    )(s0, deg)


def _stats_tail(i, zm, zm_ref, ss_ref, sq_ref):
    zm_ref[...] = zm

    @pl.when(i == 0)
    def _():
        ss_ref[...] = jnp.zeros_like(ss_ref)
        sq_ref[...] = jnp.zeros_like(sq_ref)

    ss_ref[...] += jnp.sum(zm, axis=0, keepdims=True)
    sq_ref[...] += jnp.sum(zm * zm, axis=0, keepdims=True)


def _stats_cheb_body(h_ref, x1_ref, s1_ref, deg_ref, msk_ref, w_ref, b_ref,
                     zm_ref, ss_ref, sq_ref):
    F = h_ref.shape[1]
    nrm = _nrm(deg_ref[...])
    h = h_ref[...]
    x2 = -2.0 * (s1_ref[...] * nrm) - h
    W = w_ref[...]
    z = (_dot(h, W[:F]) + _dot(x1_ref[...], W[F:2 * F])
         + _dot(x2, W[2 * F:]) + b_ref[...])
    _stats_tail(pl.program_id(0), z * msk_ref[...], zm_ref, ss_ref, sq_ref)


def _tc_stats_cheb(h, x1, s1, deg, msk, W, b):
    F = h.shape[1]
    s64 = jax.ShapeDtypeStruct((1, HID), jnp.float32)
    return pl.pallas_call(
        _stats_cheb_body, grid=(GRID,),
        in_specs=[_row_spec(F), _row_spec(F), _row_spec(F), _row_spec(1),
                  _row_spec(1), _fix_spec(3 * F, HID), _fix_spec(1, HID)],
        out_specs=[_row_spec(HID), _fix_spec(1, HID), _fix_spec(1, HID)],
        out_shape=[jax.ShapeDtypeStruct((NP, HID), jnp.float32), s64, s64],
    )(h, x1, s1, deg, msk, W, b.reshape(1, HID))


def _stats_edge_body(m_ref, c_ref, deg_ref, msk_ref, zm_ref, ss_ref, sq_ref):
    z = jnp.where(deg_ref[...] > 0.0, m_ref[...] + c_ref[...], 0.0)
    _stats_tail(pl.program_id(0), z * msk_ref[...], zm_ref, ss_ref, sq_ref)


def _tc_stats_edge(m, c, deg, msk):
    s64 = jax.ShapeDtypeStruct((1, HID), jnp.float32)
    return pl.pallas_call(
        _stats_edge_body, grid=(GRID,),
        in_specs=[_row_spec(HID), _row_spec(HID), _row_spec(1), _row_spec(1)],
        out_specs=[_row_spec(HID), _fix_spec(1, HID), _fix_spec(1, HID)],
        out_shape=[jax.ShapeDtypeStruct((NP, HID), jnp.float32), s64, s64],
    )(m, c, deg, msk)


def _bn_h(zm_ref, ss_ref, sq_ref, g_ref, b_ref, msk_ref):
    m = ss_ref[...] * (1.0 / N)
    v = sq_ref[...] * (1.0 / N) - m * m
    h = (zm_ref[...] - m) * lax.rsqrt(v + EPS) * g_ref[...] + b_ref[...]
    return jnp.maximum(h, 0.0) * msk_ref[...]


def _apply_edge_body(zm_ref, ss_ref, sq_ref, g_ref, b_ref, msk_ref,
                     tw_ref, pw_ref, tb_ref, pb_ref, h_ref, a_ref, c_ref):
    h = _bn_h(zm_ref, ss_ref, sq_ref, g_ref, b_ref, msk_ref)
    a = _dot(h, tw_ref[...])
    h_ref[...] = h
    a_ref[...] = a
    c_ref[...] = _dot(h, pw_ref[...]) - a + tb_ref[...] + pb_ref[...]


def _tc_apply_edge(zm, ss, sq, g, b, msk, tw, pw, tb, pb):
    sh = jax.ShapeDtypeStruct((NP, HID), jnp.float32)
    return pl.pallas_call(
        _apply_edge_body, grid=(GRID,),
        in_specs=[_row_spec(HID), _fix_spec(1, HID), _fix_spec(1, HID),
                  _fix_spec(1, HID), _fix_spec(1, HID), _row_spec(1),
                  _fix_spec(HID, HID), _fix_spec(HID, HID),
                  _fix_spec(1, HID), _fix_spec(1, HID)],
        out_specs=[_row_spec(HID), _row_spec(HID), _row_spec(HID)],
        out_shape=[sh, sh, sh],
    )(zm, ss, sq, g.reshape(1, HID), b.reshape(1, HID), msk, tw, pw,
      tb.reshape(1, HID), pb.reshape(1, HID))


def _apply_cheb_body(zm_ref, ss_ref, sq_ref, g_ref, b_ref, msk_ref, deg_ref,
                     h_ref, y_ref):
    h = _bn_h(zm_ref, ss_ref, sq_ref, g_ref, b_ref, msk_ref)
    h_ref[...] = h
    y_ref[...] = h * _nrm(deg_ref[...])


def _tc_apply_cheb(zm, ss, sq, g, b, msk, deg):
    sh = jax.ShapeDtypeStruct((NP, HID), jnp.float32)
    return pl.pallas_call(
        _apply_cheb_body, grid=(GRID,),
        in_specs=[_row_spec(HID), _fix_spec(1, HID), _fix_spec(1, HID),
                  _fix_spec(1, HID), _fix_spec(1, HID), _row_spec(1),
                  _row_spec(1)],
        out_specs=[_row_spec(HID), _row_spec(HID)],
        out_shape=[sh, sh],
    )(zm, ss, sq, g.reshape(1, HID), b.reshape(1, HID), msk, deg)


def _apply_last_body(zm_ref, ss_ref, sq_ref, g_ref, b_ref, msk_ref, o_ref):
    h = _bn_h(zm_ref, ss_ref, sq_ref, g_ref, b_ref, msk_ref)
    i = pl.program_id(0)

    @pl.when(i == 0)
    def _():
        o_ref[...] = jnp.zeros_like(o_ref)

    o_ref[...] += jnp.sum(h, axis=0, keepdims=True) * (1.0 / N)


def _tc_apply_last(zm, ss, sq, g, b, msk):
    return pl.pallas_call(
        _apply_last_body, grid=(GRID,),
        in_specs=[_row_spec(HID), _fix_spec(1, HID), _fix_spec(1, HID),
                  _fix_spec(1, HID), _fix_spec(1, HID), _row_spec(1)],
        out_specs=_fix_spec(1, HID),
        out_shape=jax.ShapeDtypeStruct((1, HID), jnp.float32),
    )(zm, ss, sq, g.reshape(1, HID), b.reshape(1, HID), msk)


def _consts():
    p = jnp.arange(NP, dtype=jnp.int32)
    k = p % R
    w = p // R
    valid = k < (N - w + 31) // 32
    invp = jnp.where(valid, 32 * k + w, 0).astype(jnp.int32)
    vmask = valid.astype(jnp.float32).reshape(NP, 1)
    return invp, vmask


def kernel(x, edge_index, W1, b1, bn1_g, bn1_b, e1_tw, e1_tb, e1_pw, e1_pb,
           bne1_g, bne1_b, W2, b2, bn2_g, bn2_b, e2_tw, e2_tb, e2_pw, e2_pb,
           bne2_g, bne2_b, W3, b3, bn3_g, bn3_b):
    src = edge_index[0]
    dst = edge_index[1]
    invp, vmask = _consts()

    bkt, cnts = _sc_bucket(src, dst)
    deg = _sc_deg(bkt, cnts).reshape(NP, 1)
    xp = _sc_xperm(x, invp)

    def cheb_props(h_or_y0_pair, W, b, h_for_w0):
        y0 = h_or_y0_pair
        s0 = _sc_prop(y0, bkt, cnts, False)
        x1, y1 = _tc_mid(s0, deg)
        s1 = _sc_prop(y1, bkt, cnts, False)
        return _tc_stats_cheb(h_for_w0, x1, s1, deg, vmask, W, b)

    y0 = _tc_scale(xp, deg)
    zm, ss, sq = cheb_props(y0, W1, b1, xp)
    h, a, c = _tc_apply_edge(zm, ss, sq, bn1_g, bn1_b, vmask,
                             e1_tw, e1_pw, e1_tb, e1_pb)
    m = _sc_prop(a, bkt, cnts, True)
    zm, ss, sq = _tc_stats_edge(m, c, deg, vmask)
    h, y = _tc_apply_cheb(zm, ss, sq, bne1_g, bne1_b, vmask, deg)
    zm, ss, sq = cheb_props(y, W2, b2, h)
    h, a, c = _tc_apply_edge(zm, ss, sq, bn2_g, bn2_b, vmask,
                             e2_tw, e2_pw, e2_tb, e2_pb)
    m = _sc_prop(a, bkt, cnts, True)
    zm, ss, sq = _tc_stats_edge(m, c, deg, vmask)
    h, y = _tc_apply_cheb(zm, ss, sq, bne2_g, bne2_b, vmask, deg)
    zm, ss, sq = cheb_props(y, W3, b3, h)
    return _tc_apply_last(zm, ss, sq, bn3_g, bn3_b, vmask)


# speedup vs baseline: 2.6129x; 1.1389x over previous
import functools

import jax
import jax.numpy as jnp
from jax import lax
from jax.experimental import pallas as pl
from jax.experimental.pallas import tpu as pltpu
from jax.experimental.pallas import tpu_sc as plsc

N = 50000
E = 800000
HID = 64
EPS = 1e-5

NW = 32
R = 1568
NP = NW * R
CHUNK = 128
NCHUNKS = E // CHUNK
CPW = (NCHUNKS + NW - 1) // NW
CAP = CPW * CHUNK

BR = 1568
GRID = NP // BR

_MESH = plsc.VectorSubcoreMesh(core_axis_name="c", subcore_axis_name="s")
_SC_PARAMS = pltpu.CompilerParams(use_tc_tiling_on_sc=False,
                                  needs_layout_passes=False)


def _wid():
    return lax.axis_index("s") * 2 + lax.axis_index("c")


_LANE = lambda: lax.broadcasted_iota(jnp.int32, (16,), 0)


def _bucket_body(src_hbm, dst_hbm, bkt_hbm, cnt_hbm,
                 sbuf, dbuf, wbuf, pkbuf, posp, posx, cntbuf, sem):
    w = _wid()
    lane0 = _LANE() == 0
    for g in range(3):
        cntbuf[pl.ds(g * 16, 16)] = jnp.zeros((16,), jnp.int32)

    def chunk(j, _):
        c = w + NW * j

        @pl.when(c < NCHUNKS)
        def _():
            start = c * CHUNK
            pltpu.sync_copy(src_hbm.at[pl.ds(start, CHUNK)], sbuf)
            pltpu.sync_copy(dst_hbm.at[pl.ds(start, CHUNK)], dbuf)
            for g in range(CHUNK // 16):
                sl = pl.ds(g * 16, 16)
                s = sbuf[sl]
                d = dbuf[sl]
                sp = (s & 31) * R + (s >> 5)
                wbuf[sl] = d & 31
                pkbuf[sl] = sp | ((d >> 5) << 16)

            def edge(e, _):
                b = wbuf[pl.ds(e, 16)][0]
                cnt = cntbuf[pl.ds(b, 16)][0]
                pos = (w * NW + b) * CAP + cnt
                posp[pl.ds(e, 16)] = jnp.full((16,), pos, jnp.int32)
                cv = cntbuf[pl.ds(b, 16)]
                cntbuf[pl.ds(b, 16)] = jnp.where(lane0, cnt + 1, cv)
                return 0

            lax.fori_loop(0, CHUNK, edge, 0)
            for g in range(CHUNK // 16):
                sl = pl.ds(g * 16, 16)
                posx[sl] = posp[sl]
            pltpu.async_copy(pkbuf, bkt_hbm.at[posx], sem).wait()
        return 0

    lax.fori_loop(0, CPW, chunk, 0)
    pltpu.sync_copy(cntbuf.at[pl.ds(0, NW)], cnt_hbm.at[pl.ds(w * NW, NW)])


def _sc_bucket(src, dst):
    return pl.kernel(
        _bucket_body,
        out_type=[jax.ShapeDtypeStruct((NW * NW * CAP,), jnp.int32),
                  jax.ShapeDtypeStruct((NW * NW,), jnp.int32)],
        mesh=_MESH,
        compiler_params=_SC_PARAMS,
        scratch_types=[pltpu.VMEM((CHUNK,), jnp.int32),
                       pltpu.VMEM((CHUNK,), jnp.int32),
                       pltpu.VMEM((CHUNK + 16,), jnp.int32),
                       pltpu.VMEM((CHUNK,), jnp.int32),
                       pltpu.VMEM((CHUNK + 16,), jnp.int32),
                       pltpu.VMEM((CHUNK,), jnp.int32),
                       pltpu.VMEM((NW + 16,), jnp.int32),
                       pltpu.SemaphoreType.DMA],
    )(src, dst)


def _deg_body(bkt_hbm, cnt_hbm, deg_hbm, cntv, pkbuf, dlbuf, acc, sem):
    w = _wid()
    lane16 = _LANE()
    z16 = jnp.zeros((16,), jnp.float32)

    def zero(i, _):
        for u in range(8):
            acc[pl.ds(i * 128 + u * 16, 16)] = z16
        return 0
    lax.fori_loop(0, R // 8, zero, 0)
    e0 = jnp.where(lane16 == 0, 1.0, 0.0).astype(jnp.float32)
    pltpu.sync_copy(cnt_hbm, cntv.at[pl.ds(0, NW * NW)])

    def writer(v, _):
        cnt = cntv[pl.ds(v * NW + w, 16)][0]

        def chunk(j, _):
            base = (v * NW + w) * CAP + j * CHUNK
            pltpu.sync_copy(bkt_hbm.at[pl.ds(base, CHUNK)], pkbuf)
            ne = jnp.minimum(cnt - j * CHUNK, CHUNK)
            for g in range(CHUNK // 16):
                sl = pl.ds(g * 16, 16)
                ok = (lane16 + g * 16) < ne
                dlbuf[sl] = jnp.where(
                    ok, jnp.clip(pkbuf[sl] >> 16, 0, R - 1), R - 1)

            def grp(g, _):
                dlv = dlbuf[pl.ds(g * 16, 16)]
                for l in range(16):
                    plsc.addupdate(acc.at[pl.ds(dlv[l] * 16, 16)], e0)
                return 0

            lax.fori_loop(0, CHUNK // 16, grp, 0)
            return 0

        lax.fori_loop(0, (cnt + CHUNK - 1) // CHUNK, chunk, 0)
        return 0

    lax.fori_loop(0, NW, writer, 0)
    pltpu.sync_copy(acc, deg_hbm.at[pl.ds(w * R * 16, R * 16)])


def _sc_deg(bkt, cnts):
    out = pl.kernel(
        _deg_body,
        out_type=jax.ShapeDtypeStruct((NP * 16,), jnp.float32),
        mesh=_MESH,
        compiler_params=_SC_PARAMS,
        scratch_types=[pltpu.VMEM((NW * NW + 16,), jnp.int32),
                       pltpu.VMEM((CHUNK,), jnp.int32),
                       pltpu.VMEM((CHUNK,), jnp.int32),
                       pltpu.VMEM((R * 16,), jnp.float32),
                       pltpu.SemaphoreType.DMA],
    )(bkt, cnts)
    return out.reshape(NP, 16)[:, 0]


def _prop_body(is_max, F, table_hbm, bkt_hbm, cnt_hbm, out_hbm,
               cntv, pk0, pk1, idx0, idx1, dl0, dl1, rows0, rows1, acc,
               sem0, sem1):
    w = _wid()
    lane16 = _LANE()
    pks, idxs, dls, rowss, sems = (pk0, pk1), (idx0, idx1), (dl0, dl1), \
        (rows0, rows1), (sem0, sem1)
    fill = jnp.full((16,), -3.4e38 if is_max else 0.0, jnp.float32)

    def zero(i, _):
        for u in range(8):
            acc[pl.ds(i * 128 + u * 16, 16)] = fill
        return 0
    lax.fori_loop(0, (R * F) // 128, zero, 0)
    pltpu.sync_copy(cnt_hbm, cntv.at[pl.ds(0, NW * NW)])

    def nchunks(vi):
        cnt = cntv[pl.ds(jnp.minimum(vi, NW - 1) * NW + w, 16)][0]
        return cnt, jnp.maximum((cnt + CHUNK - 1) // CHUNK, 1)

    def total(vi, t):
        return t + nchunks(vi)[1]
    T = lax.fori_loop(0, NW, total, 0)

    def issue(vi, ji, slot):
        cnt, _ = nchunks(vi)
        ne = jnp.clip(cnt - ji * CHUNK, 0, CHUNK)
        base = (vi * NW + w) * CAP + ji * CHUNK
        pltpu.sync_copy(bkt_hbm.at[pl.ds(base, CHUNK)], pks[slot])
        for g in range(CHUNK // 16):
            sl = pl.ds(g * 16, 16)
            pk = pks[slot][sl]
            ok = (lane16 + g * 16) < ne
            idxs[slot][sl] = jnp.minimum(pk & 0xFFFF, NP - 1)
            dls[slot][sl] = jnp.where(ok, jnp.clip(pk >> 16, 0, R - 1),
                                      R - 1)
        pltpu.async_copy(table_hbm.at[idxs[slot]], rowss[slot], sems[slot])

    def adv(c, vi, ji):
        cnt, nch = nchunks(vi)
        more = (ji + 1) < nch
        nvi = jnp.where(more, vi, vi + 1)
        nji = jnp.where(more, ji + 1, 0)
        return jnp.where(c, nvi, vi), jnp.where(c, nji, ji)

    def wait(slot):
        pltpu.make_async_copy(table_hbm.at[idxs[slot]], rowss[slot],
                              sems[slot]).wait()

    def process(slot):
        def grp(g, _):
            dlv = dls[slot][pl.ds(g * 16, 16)]
            for l in range(16):
                b = dlv[l] * F
                for q in range(F // 16):
                    sl = pl.ds(b + q * 16, 16)
                    r = rowss[slot][g * 16 + l, pl.ds(q * 16, 16)]
                    if is_max:
                        acc[sl] = jnp.maximum(acc[sl], r)
                    else:
                        plsc.addupdate(acc.at[sl], r)
            return 0
        lax.fori_loop(0, CHUNK // 16, grp, 0)

    issue(0, 0, 0)

    def step2(u, st):
        vi, ji = st
        t1 = 2 * u + 1

        @pl.when(t1 < T)
        def _():
            issue(vi, ji, 1)
        vi, ji = adv(t1 < T, vi, ji)
        wait(0)
        process(0)

        @pl.when(t1 < T)
        def _():
            @pl.when(t1 + 1 < T)
            def _():
                issue(vi, ji, 0)
            wait(1)
            process(1)
        vi, ji = adv(t1 + 1 < T, vi, ji)
        return (vi, ji)

    lax.fori_loop(0, (T + 1) // 2, step2, adv(True, 0, 0))
    pltpu.sync_copy(acc, out_hbm.at[pl.ds(w * R * F, R * F)])


def _sc_prop(table, bkt, cnts, is_max):
    F = table.shape[1]
    out = pl.kernel(
        functools.partial(_prop_body, is_max, F),
        out_type=jax.ShapeDtypeStruct((NP * F,), jnp.float32),
        mesh=_MESH,
        compiler_params=_SC_PARAMS,
        scratch_types=[pltpu.VMEM((NW * NW + 16,), jnp.int32)]
        + [pltpu.VMEM((CHUNK,), jnp.int32)] * 4
        + [pltpu.VMEM((CHUNK,), jnp.int32)] * 2
        + [pltpu.VMEM((CHUNK, F), jnp.float32)] * 2
        + [pltpu.VMEM((R * F,), jnp.float32),
           pltpu.SemaphoreType.DMA, pltpu.SemaphoreType.DMA],
    )(table, bkt, cnts)
    return out.reshape(NP, F)


def _xperm_body(x_hbm, inv_hbm, out_hbm, idxbuf, rows, sem):
    w = _wid()
    nfull = R // CHUNK
    tail = R - nfull * CHUNK
    for j in range(nfull):
        pltpu.sync_copy(inv_hbm.at[pl.ds(w * R + j * CHUNK, CHUNK)], idxbuf)
        pltpu.async_copy(x_hbm.at[idxbuf], rows, sem).wait()
        pltpu.sync_copy(rows, out_hbm.at[pl.ds(w * R + j * CHUNK, CHUNK)])
    pltpu.sync_copy(inv_hbm.at[pl.ds(w * R + nfull * CHUNK, tail)],
                    idxbuf.at[pl.ds(0, tail)])
    pltpu.async_copy(x_hbm.at[idxbuf.at[pl.ds(0, tail)]],
                     rows.at[pl.ds(0, tail)], sem).wait()
    pltpu.sync_copy(rows.at[pl.ds(0, tail)],
                    out_hbm.at[pl.ds(w * R + nfull * CHUNK, tail)])


def _sc_xperm(x, invp):
    F = x.shape[1]
    return pl.kernel(
        _xperm_body,
        out_type=jax.ShapeDtypeStruct((NP, F), jnp.float32),
        mesh=_MESH,
        compiler_params=_SC_PARAMS,
        scratch_types=[pltpu.VMEM((CHUNK,), jnp.int32),
                       pltpu.VMEM((CHUNK, F), jnp.float32),
                       pltpu.SemaphoreType.DMA],
    )(x, invp)


def _nrm(deg):
    return lax.rsqrt(jnp.clip(deg, 1.0, None))


def _dot(a, b):
    return jax.lax.dot_general(a, b, (((1,), (0,)), ((), ())),
                               precision=jax.lax.Precision.HIGHEST)


def _row_spec(F):
    return pl.BlockSpec((BR, F), lambda i: (i, 0))


def _fix_spec(r, c):
    return pl.BlockSpec((r, c), lambda i: (0, 0))


def _scale_body(x_ref, deg_ref, y_ref):
    y_ref[...] = x_ref[...] * _nrm(deg_ref[...])


def _tc_scale(x, deg):
    F = x.shape[1]
    return pl.pallas_call(
        _scale_body, grid=(GRID,),
        in_specs=[_row_spec(F), _row_spec(1)],
        out_specs=_row_spec(F),
        out_shape=jax.ShapeDtypeStruct((NP, F), jnp.float32),
    )(x, deg)


def _mid_body(s_ref, deg_ref, x1_ref, y1_ref):
    nrm = _nrm(deg_ref[...])
    x1 = -(s_ref[...] * nrm)
    x1_ref[...] = x1
    y1_ref[...] = x1 * nrm


def _tc_mid(s0, deg):
    F = s0.shape[1]
    sh = jax.ShapeDtypeStruct((NP, F), jnp.float32)
    return pl.pallas_call(
        _mid_body, grid=(GRID,),
        in_specs=[_row_spec(F), _row_spec(1)],
        out_specs=[_row_spec(F), _row_spec(F)],
        out_shape=[sh, sh],
    )(s0, deg)


def _stats_tail(i, zm, zm_ref, ss_ref, sq_ref):
    zm_ref[...] = zm

    @pl.when(i == 0)
    def _():
        ss_ref[...] = jnp.zeros_like(ss_ref)
        sq_ref[...] = jnp.zeros_like(sq_ref)

    ss_ref[...] += jnp.sum(zm, axis=0, keepdims=True)
    sq_ref[...] += jnp.sum(zm * zm, axis=0, keepdims=True)


def _stats_cheb_body(h_ref, x1_ref, s1_ref, deg_ref, msk_ref, w_ref, b_ref,
                     zm_ref, ss_ref, sq_ref):
    F = h_ref.shape[1]
    nrm = _nrm(deg_ref[...])
    h = h_ref[...]
    x2 = -2.0 * (s1_ref[...] * nrm) - h
    W = w_ref[...]
    z = (_dot(h, W[:F]) + _dot(x1_ref[...], W[F:2 * F])
         + _dot(x2, W[2 * F:]) + b_ref[...])
    _stats_tail(pl.program_id(0), z * msk_ref[...], zm_ref, ss_ref, sq_ref)


def _tc_stats_cheb(h, x1, s1, deg, msk, W, b):
    F = h.shape[1]
    s64 = jax.ShapeDtypeStruct((1, HID), jnp.float32)
    return pl.pallas_call(
        _stats_cheb_body, grid=(GRID,),
        in_specs=[_row_spec(F), _row_spec(F), _row_spec(F), _row_spec(1),
                  _row_spec(1), _fix_spec(3 * F, HID), _fix_spec(1, HID)],
        out_specs=[_row_spec(HID), _fix_spec(1, HID), _fix_spec(1, HID)],
        out_shape=[jax.ShapeDtypeStruct((NP, HID), jnp.float32), s64, s64],
    )(h, x1, s1, deg, msk, W, b.reshape(1, HID))


def _stats_edge_body(m_ref, c_ref, deg_ref, msk_ref, zm_ref, ss_ref, sq_ref):
    z = jnp.where(deg_ref[...] > 0.0, m_ref[...] + c_ref[...], 0.0)
    _stats_tail(pl.program_id(0), z * msk_ref[...], zm_ref, ss_ref, sq_ref)


def _tc_stats_edge(m, c, deg, msk):
    s64 = jax.ShapeDtypeStruct((1, HID), jnp.float32)
    return pl.pallas_call(
        _stats_edge_body, grid=(GRID,),
        in_specs=[_row_spec(HID), _row_spec(HID), _row_spec(1), _row_spec(1)],
        out_specs=[_row_spec(HID), _fix_spec(1, HID), _fix_spec(1, HID)],
        out_shape=[jax.ShapeDtypeStruct((NP, HID), jnp.float32), s64, s64],
    )(m, c, deg, msk)


def _bn_h(zm_ref, ss_ref, sq_ref, g_ref, b_ref, msk_ref):
    m = ss_ref[...] * (1.0 / N)
    v = sq_ref[...] * (1.0 / N) - m * m
    h = (zm_ref[...] - m) * lax.rsqrt(v + EPS) * g_ref[...] + b_ref[...]
    return jnp.maximum(h, 0.0) * msk_ref[...]


def _apply_edge_body(zm_ref, ss_ref, sq_ref, g_ref, b_ref, msk_ref,
                     tw_ref, pw_ref, tb_ref, pb_ref, h_ref, a_ref, c_ref):
    h = _bn_h(zm_ref, ss_ref, sq_ref, g_ref, b_ref, msk_ref)
    a = _dot(h, tw_ref[...])
    h_ref[...] = h
    a_ref[...] = a
    c_ref[...] = _dot(h, pw_ref[...]) - a + tb_ref[...] + pb_ref[...]


def _tc_apply_edge(zm, ss, sq, g, b, msk, tw, pw, tb, pb):
    sh = jax.ShapeDtypeStruct((NP, HID), jnp.float32)
    return pl.pallas_call(
        _apply_edge_body, grid=(GRID,),
        in_specs=[_row_spec(HID), _fix_spec(1, HID), _fix_spec(1, HID),
                  _fix_spec(1, HID), _fix_spec(1, HID), _row_spec(1),
                  _fix_spec(HID, HID), _fix_spec(HID, HID),
                  _fix_spec(1, HID), _fix_spec(1, HID)],
        out_specs=[_row_spec(HID), _row_spec(HID), _row_spec(HID)],
        out_shape=[sh, sh, sh],
    )(zm, ss, sq, g.reshape(1, HID), b.reshape(1, HID), msk, tw, pw,
      tb.reshape(1, HID), pb.reshape(1, HID))


def _apply_cheb_body(zm_ref, ss_ref, sq_ref, g_ref, b_ref, msk_ref, deg_ref,
                     h_ref, y_ref):
    h = _bn_h(zm_ref, ss_ref, sq_ref, g_ref, b_ref, msk_ref)
    h_ref[...] = h
    y_ref[...] = h * _nrm(deg_ref[...])


def _tc_apply_cheb(zm, ss, sq, g, b, msk, deg):
    sh = jax.ShapeDtypeStruct((NP, HID), jnp.float32)
    return pl.pallas_call(
        _apply_cheb_body, grid=(GRID,),
        in_specs=[_row_spec(HID), _fix_spec(1, HID), _fix_spec(1, HID),
                  _fix_spec(1, HID), _fix_spec(1, HID), _row_spec(1),
                  _row_spec(1)],
        out_specs=[_row_spec(HID), _row_spec(HID)],
        out_shape=[sh, sh],
    )(zm, ss, sq, g.reshape(1, HID), b.reshape(1, HID), msk, deg)


def _apply_last_body(zm_ref, ss_ref, sq_ref, g_ref, b_ref, msk_ref, o_ref):
    h = _bn_h(zm_ref, ss_ref, sq_ref, g_ref, b_ref, msk_ref)
    i = pl.program_id(0)

    @pl.when(i == 0)
    def _():
        o_ref[...] = jnp.zeros_like(o_ref)

    o_ref[...] += jnp.sum(h, axis=0, keepdims=True) * (1.0 / N)


def _tc_apply_last(zm, ss, sq, g, b, msk):
    return pl.pallas_call(
        _apply_last_body, grid=(GRID,),
        in_specs=[_row_spec(HID), _fix_spec(1, HID), _fix_spec(1, HID),
                  _fix_spec(1, HID), _fix_spec(1, HID), _row_spec(1)],
        out_specs=_fix_spec(1, HID),
        out_shape=jax.ShapeDtypeStruct((1, HID), jnp.float32),
    )(zm, ss, sq, g.reshape(1, HID), b.reshape(1, HID), msk)


def _consts():
    p = jnp.arange(NP, dtype=jnp.int32)
    k = p % R
    w = p // R
    valid = k < (N - w + 31) // 32
    invp = jnp.where(valid, 32 * k + w, 0).astype(jnp.int32)
    vmask = valid.astype(jnp.float32).reshape(NP, 1)
    return invp, vmask


def kernel(x, edge_index, W1, b1, bn1_g, bn1_b, e1_tw, e1_tb, e1_pw, e1_pb,
           bne1_g, bne1_b, W2, b2, bn2_g, bn2_b, e2_tw, e2_tb, e2_pw, e2_pb,
           bne2_g, bne2_b, W3, b3, bn3_g, bn3_b):
    src = edge_index[0]
    dst = edge_index[1]
    invp, vmask = _consts()

    bkt, cnts = _sc_bucket(src, dst)
    deg = _sc_deg(bkt, cnts).reshape(NP, 1)
    xp = _sc_xperm(x, invp)

    def cheb_props(h_or_y0_pair, W, b, h_for_w0):
        y0 = h_or_y0_pair
        s0 = _sc_prop(y0, bkt, cnts, False)
        x1, y1 = _tc_mid(s0, deg)
        s1 = _sc_prop(y1, bkt, cnts, False)
        return _tc_stats_cheb(h_for_w0, x1, s1, deg, vmask, W, b)

    y0 = _tc_scale(xp, deg)
    zm, ss, sq = cheb_props(y0, W1, b1, xp)
    h, a, c = _tc_apply_edge(zm, ss, sq, bn1_g, bn1_b, vmask,
                             e1_tw, e1_pw, e1_tb, e1_pb)
    m = _sc_prop(a, bkt, cnts, True)
    zm, ss, sq = _tc_stats_edge(m, c, deg, vmask)
    h, y = _tc_apply_cheb(zm, ss, sq, bne1_g, bne1_b, vmask, deg)
    zm, ss, sq = cheb_props(y, W2, b2, h)
    h, a, c = _tc_apply_edge(zm, ss, sq, bn2_g, bn2_b, vmask,
                             e2_tw, e2_pw, e2_tb, e2_pb)
    m = _sc_prop(a, bkt, cnts, True)
    zm, ss, sq = _tc_stats_edge(m, c, deg, vmask)
    h, y = _tc_apply_cheb(zm, ss, sq, bne2_g, bne2_b, vmask, deg)
    zm, ss, sq = cheb_props(y, W3, b3, h)
    return _tc_apply_last(zm, ss, sq, bn3_g, bn3_b, vmask)
